# baseline (speedup 1.0000x reference)
import functools

import jax
import jax.numpy as jnp
from jax import lax
from jax.experimental import pallas as pl
from jax.experimental.pallas import tpu as pltpu
from jax.experimental.pallas import tpu_sc as plsc

_B, _S, _H, _V = 32, 512, 768, 30522
_EPS = 1e-12
_NC = 2
_NW = 32
_K = 8
_BK = _B // _K
_R = _BK * _S
_RPW = _R // _NW
_CH = 32
_NCH = _RPW // _CH


def _gather_phase(ids_hbm, word_hbm, inter_hbm, ids_v, w0, w1,
                  sg0, sg1, ss0, ss1):
    cid = lax.axis_index("c")
    sid = lax.axis_index("s")
    wid = sid * _NC + cid
    r0 = wid * _RPW

    pltpu.sync_copy(ids_hbm.at[pl.ds(r0, _RPW)], ids_v)

    def gather(c, buf, sem):
        return pltpu.make_async_copy(
            word_hbm.at[ids_v.at[pl.ds(c * _CH, _CH)]], buf, sem)

    def store(c, buf, sem):
        return pltpu.make_async_copy(
            buf, inter_hbm.at[pl.ds(r0 + c * _CH, _CH)], sem)

    gather(0, w0, sg0).start()
    gather(1, w1, sg1).start()

    def step(s, _):
        c0 = 2 * s
        c1 = c0 + 1
        gather(c0, w0, sg0).wait()
        store(c0, w0, ss0).start()
        gather(c1, w1, sg1).wait()
        store(c1, w1, ss1).start()
        store(c0, w0, ss0).wait()

        @pl.when(c0 + 2 < _NCH)
        def _():
            gather(c0 + 2, w0, sg0).start()

        store(c1, w1, ss1).wait()

        @pl.when(c1 + 2 < _NCH)
        def _():
            gather(c1 + 2, w1, sg1).start()

        return 0

    lax.fori_loop(0, _NCH // 2, step, 0)


@functools.partial(
    pl.kernel,
    out_type=jax.ShapeDtypeStruct((_R, _H), jnp.float32),
    mesh=plsc.VectorSubcoreMesh(core_axis_name="c", subcore_axis_name="s"),
    scratch_types=[
        pltpu.VMEM((_RPW,), jnp.int32),
        pltpu.VMEM((_CH, _H), jnp.float32),
        pltpu.VMEM((_CH, _H), jnp.float32),
        pltpu.SemaphoreType.DMA,
        pltpu.SemaphoreType.DMA,
        pltpu.SemaphoreType.DMA,
        pltpu.SemaphoreType.DMA,
    ],
)
def _sc_gather(ids_hbm, word_hbm, inter_hbm, *scratch):
    _gather_phase(ids_hbm, word_hbm, inter_hbm, *scratch)


_BR = 1024


def _ln_body(inter_ref, tt_ref, vv_ref, pos_ref, table_ref,
             gamma_ref, beta_ref, o_ref):
    tt = tt_ref[...]
    vv = vv_ref[...]
    iota8 = lax.broadcasted_iota(jnp.int32, (_BR, 8), 1)
    onehot = ((iota8 == tt[:, None]) | (iota8 == vv[:, None] + 2))
    combo = jnp.dot(onehot.astype(jnp.float32), table_ref[...],
                    preferred_element_type=jnp.float32)
    pos = pos_ref[...]
    x = inter_ref[...] + combo
    x = (x.reshape(_BR // _S, _S, _H) + pos[None]).reshape(_BR, _H)
    mean = jnp.mean(x, axis=-1, keepdims=True)
    xc = x - mean
    var = jnp.mean(xc * xc, axis=-1, keepdims=True)
    o_ref[...] = (xc * lax.rsqrt(var + _EPS)) * gamma_ref[...] + beta_ref[...]


def _ln_first(inter_ref, tt_ref, vv_ref, pos_ref, table_ref,
              gamma_ref, beta_ref, o_ref):
    _ln_body(inter_ref, tt_ref, vv_ref, pos_ref, table_ref,
             gamma_ref, beta_ref, o_ref)


def _ln_chained(buf_ref, inter_ref, tt_ref, vv_ref, pos_ref, table_ref,
                gamma_ref, beta_ref, o_ref):
    del buf_ref
    _ln_body(inter_ref, tt_ref, vv_ref, pos_ref, table_ref,
             gamma_ref, beta_ref, o_ref)


_data_specs = [
    pl.BlockSpec((_BR, _H), lambda i: (i, 0)),
    pl.BlockSpec((_BR,), lambda i: (i,)),
    pl.BlockSpec((_BR,), lambda i: (i,)),
    pl.BlockSpec((_S, _H), lambda i: (0, 0)),
    pl.BlockSpec((8, _H), lambda i: (0, 0)),
    pl.BlockSpec((1, _H), lambda i: (0, 0)),
    pl.BlockSpec((1, _H), lambda i: (0, 0)),
]

_out_shape = jax.ShapeDtypeStruct((_B * _S, _H), jnp.float32)
_GB = _R // _BR


def _make_ln_call(k):
    out_spec = pl.BlockSpec((_BR, _H), lambda i, k=k: (i + k * _GB, 0))
    if k == 0:
        return pl.pallas_call(
            _ln_first, grid=(_GB,), in_specs=_data_specs,
            out_specs=out_spec, out_shape=_out_shape)
    return pl.pallas_call(
        _ln_chained, grid=(_GB,),
        in_specs=[pl.BlockSpec(memory_space=pl.ANY)] + _data_specs,
        out_specs=out_spec, out_shape=_out_shape,
        input_output_aliases={0: 0})


_ln_calls = [_make_ln_call(k) for k in range(_K)]


def kernel(input_ids, token_type_ids, variant_ids, word_emb, pos_emb,
           type_emb, variant_emb, gamma, beta):
    ids = input_ids.astype(jnp.int32).reshape(-1)
    tt = token_type_ids.astype(jnp.int32).reshape(-1)
    vv = variant_ids.astype(jnp.int32).reshape(-1)
    table = jnp.concatenate([type_emb, variant_emb], axis=0)
    g = gamma.reshape(1, _H)
    b = beta.reshape(1, _H)

    inters = [_sc_gather(ids[k * _R:(k + 1) * _R], word_emb)
              for k in range(_K)]

    buf = None
    for k in range(_K):
        args = (inters[k], tt[k * _R:(k + 1) * _R], vv[k * _R:(k + 1) * _R],
                pos_emb, table, g, b)
        buf = _ln_calls[k](*args) if k == 0 else _ln_calls[k](buf, *args)
    return buf.reshape(_B, _S, _H)


# speedup vs baseline: 1.1511x; 1.1511x over previous
import functools

import jax
import jax.numpy as jnp
from jax import lax
from jax.experimental import pallas as pl
from jax.experimental.pallas import tpu as pltpu
from jax.experimental.pallas import tpu_sc as plsc

_B, _S, _H, _V = 32, 512, 768, 30522
_EPS = 1e-12
_NC = 2
_NW = 32
_K = 2
_BK = _B // _K
_R = _BK * _S
_RPW = _R // _NW
_CH = 32
_NCH = _RPW // _CH


def _gather_phase(ids_hbm, word_hbm, inter_hbm, ids_v, w0, w1,
                  sg0, sg1, ss0, ss1):
    cid = lax.axis_index("c")
    sid = lax.axis_index("s")
    wid = sid * _NC + cid
    r0 = wid * _RPW

    pltpu.sync_copy(ids_hbm.at[pl.ds(r0, _RPW)], ids_v)

    def gather(c, buf, sem):
        return pltpu.make_async_copy(
            word_hbm.at[ids_v.at[pl.ds(c * _CH, _CH)]], buf, sem)

    def store(c, buf, sem):
        return pltpu.make_async_copy(
            buf, inter_hbm.at[pl.ds(r0 + c * _CH, _CH)], sem)

    gather(0, w0, sg0).start()
    gather(1, w1, sg1).start()

    def step(s, _):
        c0 = 2 * s
        c1 = c0 + 1
        gather(c0, w0, sg0).wait()
        store(c0, w0, ss0).start()
        gather(c1, w1, sg1).wait()
        store(c1, w1, ss1).start()
        store(c0, w0, ss0).wait()

        @pl.when(c0 + 2 < _NCH)
        def _():
            gather(c0 + 2, w0, sg0).start()

        store(c1, w1, ss1).wait()

        @pl.when(c1 + 2 < _NCH)
        def _():
            gather(c1 + 2, w1, sg1).start()

        return 0

    lax.fori_loop(0, _NCH // 2, step, 0)


@functools.partial(
    pl.kernel,
    out_type=jax.ShapeDtypeStruct((_R, _H), jnp.float32),
    mesh=plsc.VectorSubcoreMesh(core_axis_name="c", subcore_axis_name="s"),
    scratch_types=[
        pltpu.VMEM((_RPW,), jnp.int32),
        pltpu.VMEM((_CH, _H), jnp.float32),
        pltpu.VMEM((_CH, _H), jnp.float32),
        pltpu.SemaphoreType.DMA,
        pltpu.SemaphoreType.DMA,
        pltpu.SemaphoreType.DMA,
        pltpu.SemaphoreType.DMA,
    ],
)
def _sc_gather(ids_hbm, word_hbm, inter_hbm, *scratch):
    _gather_phase(ids_hbm, word_hbm, inter_hbm, *scratch)


_BR = 1024


def _ln_body(inter_ref, tt_ref, vv_ref, pos_ref, table_ref,
             gamma_ref, beta_ref, o_ref):
    tt = tt_ref[...]
    vv = vv_ref[...]
    iota8 = lax.broadcasted_iota(jnp.int32, (_BR, 8), 1)
    onehot = ((iota8 == tt[:, None]) | (iota8 == vv[:, None] + 2))
    combo = jnp.dot(onehot.astype(jnp.float32), table_ref[...],
                    preferred_element_type=jnp.float32)
    pos = pos_ref[...]
    x = inter_ref[...] + combo
    x = (x.reshape(_BR // _S, _S, _H) + pos[None]).reshape(_BR, _H)
    mean = jnp.mean(x, axis=-1, keepdims=True)
    xc = x - mean
    var = jnp.mean(xc * xc, axis=-1, keepdims=True)
    o_ref[...] = (xc * lax.rsqrt(var + _EPS)) * gamma_ref[...] + beta_ref[...]


def _ln_first(inter_ref, tt_ref, vv_ref, pos_ref, table_ref,
              gamma_ref, beta_ref, o_ref):
    _ln_body(inter_ref, tt_ref, vv_ref, pos_ref, table_ref,
             gamma_ref, beta_ref, o_ref)


def _ln_chained(buf_ref, inter_ref, tt_ref, vv_ref, pos_ref, table_ref,
                gamma_ref, beta_ref, o_ref):
    del buf_ref
    _ln_body(inter_ref, tt_ref, vv_ref, pos_ref, table_ref,
             gamma_ref, beta_ref, o_ref)


_data_specs = [
    pl.BlockSpec((_BR, _H), lambda i: (i, 0)),
    pl.BlockSpec((_BR,), lambda i: (i,)),
    pl.BlockSpec((_BR,), lambda i: (i,)),
    pl.BlockSpec((_S, _H), lambda i: (0, 0)),
    pl.BlockSpec((8, _H), lambda i: (0, 0)),
    pl.BlockSpec((1, _H), lambda i: (0, 0)),
    pl.BlockSpec((1, _H), lambda i: (0, 0)),
]

_out_shape = jax.ShapeDtypeStruct((_B * _S, _H), jnp.float32)
_GB = _R // _BR


def _make_ln_call(k):
    out_spec = pl.BlockSpec((_BR, _H), lambda i, k=k: (i + k * _GB, 0))
    if k == 0:
        return pl.pallas_call(
            _ln_first, grid=(_GB,), in_specs=_data_specs,
            out_specs=out_spec, out_shape=_out_shape)
    return pl.pallas_call(
        _ln_chained, grid=(_GB,),
        in_specs=[pl.BlockSpec(memory_space=pl.ANY)] + _data_specs,
        out_specs=out_spec, out_shape=_out_shape,
        input_output_aliases={0: 0})


_ln_calls = [_make_ln_call(k) for k in range(_K)]


def kernel(input_ids, token_type_ids, variant_ids, word_emb, pos_emb,
           type_emb, variant_emb, gamma, beta):
    ids = input_ids.astype(jnp.int32).reshape(-1)
    tt = token_type_ids.astype(jnp.int32).reshape(-1)
    vv = variant_ids.astype(jnp.int32).reshape(-1)
    table = jnp.concatenate([type_emb, variant_emb], axis=0)
    g = gamma.reshape(1, _H)
    b = beta.reshape(1, _H)

    inters = [_sc_gather(ids[k * _R:(k + 1) * _R], word_emb)
              for k in range(_K)]

    buf = None
    for k in range(_K):
        args = (inters[k], tt[k * _R:(k + 1) * _R], vv[k * _R:(k + 1) * _R],
                pos_emb, table, g, b)
        buf = _ln_calls[k](*args) if k == 0 else _ln_calls[k](buf, *args)
    return buf.reshape(_B, _S, _H)


# speedup vs baseline: 1.1638x; 1.0111x over previous
import functools

import jax
import jax.numpy as jnp
from jax import lax
from jax.experimental import pallas as pl
from jax.experimental.pallas import tpu as pltpu
from jax.experimental.pallas import tpu_sc as plsc

_B, _S, _H, _V = 32, 512, 768, 30522
_EPS = 1e-12
_NC = 2
_NW = 32
_K = 2
_BK = _B // _K
_R = _BK * _S
_RPW = _R // _NW
_CH = 64
_NCH = _RPW // _CH


def _gather_phase(ids_hbm, word_hbm, inter_hbm, ids_v, w0, w1,
                  sg0, sg1, ss0, ss1):
    cid = lax.axis_index("c")
    sid = lax.axis_index("s")
    wid = sid * _NC + cid
    r0 = wid * _RPW

    pltpu.sync_copy(ids_hbm.at[pl.ds(r0, _RPW)], ids_v)

    def gather(c, buf, sem):
        return pltpu.make_async_copy(
            word_hbm.at[ids_v.at[pl.ds(c * _CH, _CH)]], buf, sem)

    def store(c, buf, sem):
        return pltpu.make_async_copy(
            buf, inter_hbm.at[pl.ds(r0 + c * _CH, _CH)], sem)

    gather(0, w0, sg0).start()
    gather(1, w1, sg1).start()

    def step(s, _):
        c0 = 2 * s
        c1 = c0 + 1
        gather(c0, w0, sg0).wait()
        store(c0, w0, ss0).start()
        gather(c1, w1, sg1).wait()
        store(c1, w1, ss1).start()
        store(c0, w0, ss0).wait()

        @pl.when(c0 + 2 < _NCH)
        def _():
            gather(c0 + 2, w0, sg0).start()

        store(c1, w1, ss1).wait()

        @pl.when(c1 + 2 < _NCH)
        def _():
            gather(c1 + 2, w1, sg1).start()

        return 0

    lax.fori_loop(0, _NCH // 2, step, 0)


@functools.partial(
    pl.kernel,
    out_type=jax.ShapeDtypeStruct((_R, _H), jnp.float32),
    mesh=plsc.VectorSubcoreMesh(core_axis_name="c", subcore_axis_name="s"),
    scratch_types=[
        pltpu.VMEM((_RPW,), jnp.int32),
        pltpu.VMEM((_CH, _H), jnp.float32),
        pltpu.VMEM((_CH, _H), jnp.float32),
        pltpu.SemaphoreType.DMA,
        pltpu.SemaphoreType.DMA,
        pltpu.SemaphoreType.DMA,
        pltpu.SemaphoreType.DMA,
    ],
)
def _sc_gather(ids_hbm, word_hbm, inter_hbm, *scratch):
    _gather_phase(ids_hbm, word_hbm, inter_hbm, *scratch)


_BR = 1024


def _ln_body(inter_ref, tt_ref, vv_ref, pos_ref, table_ref,
             gamma_ref, beta_ref, o_ref):
    tt = tt_ref[...]
    vv = vv_ref[...]
    iota8 = lax.broadcasted_iota(jnp.int32, (_BR, 8), 1)
    onehot = ((iota8 == tt[:, None]) | (iota8 == vv[:, None] + 2))
    combo = jnp.dot(onehot.astype(jnp.float32), table_ref[...],
                    preferred_element_type=jnp.float32)
    pos = pos_ref[...]
    x = inter_ref[...] + combo
    x = (x.reshape(_BR // _S, _S, _H) + pos[None]).reshape(_BR, _H)
    mean = jnp.mean(x, axis=-1, keepdims=True)
    xc = x - mean
    var = jnp.mean(xc * xc, axis=-1, keepdims=True)
    o_ref[...] = (xc * lax.rsqrt(var + _EPS)) * gamma_ref[...] + beta_ref[...]


def _ln_first(inter_ref, tt_ref, vv_ref, pos_ref, table_ref,
              gamma_ref, beta_ref, o_ref):
    _ln_body(inter_ref, tt_ref, vv_ref, pos_ref, table_ref,
             gamma_ref, beta_ref, o_ref)


def _ln_chained(buf_ref, inter_ref, tt_ref, vv_ref, pos_ref, table_ref,
                gamma_ref, beta_ref, o_ref):
    del buf_ref
    _ln_body(inter_ref, tt_ref, vv_ref, pos_ref, table_ref,
             gamma_ref, beta_ref, o_ref)


_data_specs = [
    pl.BlockSpec((_BR, _H), lambda i: (i, 0)),
    pl.BlockSpec((_BR,), lambda i: (i,)),
    pl.BlockSpec((_BR,), lambda i: (i,)),
    pl.BlockSpec((_S, _H), lambda i: (0, 0)),
    pl.BlockSpec((8, _H), lambda i: (0, 0)),
    pl.BlockSpec((1, _H), lambda i: (0, 0)),
    pl.BlockSpec((1, _H), lambda i: (0, 0)),
]

_out_shape = jax.ShapeDtypeStruct((_B * _S, _H), jnp.float32)
_GB = _R // _BR


def _make_ln_call(k):
    out_spec = pl.BlockSpec((_BR, _H), lambda i, k=k: (i + k * _GB, 0))
    if k == 0:
        return pl.pallas_call(
            _ln_first, grid=(_GB,), in_specs=_data_specs,
            out_specs=out_spec, out_shape=_out_shape)
    return pl.pallas_call(
        _ln_chained, grid=(_GB,),
        in_specs=[pl.BlockSpec(memory_space=pl.ANY)] + _data_specs,
        out_specs=out_spec, out_shape=_out_shape,
        input_output_aliases={0: 0})


_ln_calls = [_make_ln_call(k) for k in range(_K)]


def kernel(input_ids, token_type_ids, variant_ids, word_emb, pos_emb,
           type_emb, variant_emb, gamma, beta):
    ids = input_ids.astype(jnp.int32).reshape(-1)
    tt = token_type_ids.astype(jnp.int32).reshape(-1)
    vv = variant_ids.astype(jnp.int32).reshape(-1)
    table = jnp.concatenate([type_emb, variant_emb], axis=0)
    g = gamma.reshape(1, _H)
    b = beta.reshape(1, _H)

    inters = [_sc_gather(ids[k * _R:(k + 1) * _R], word_emb)
              for k in range(_K)]

    buf = None
    for k in range(_K):
        args = (inters[k], tt[k * _R:(k + 1) * _R], vv[k * _R:(k + 1) * _R],
                pos_emb, table, g, b)
        buf = _ln_calls[k](*args) if k == 0 else _ln_calls[k](buf, *args)
    return buf.reshape(_B, _S, _H)
